# baseline (speedup 1.0000x reference)
import functools

import jax
import jax.numpy as jnp
from jax import lax
from jax.experimental import pallas as pl
from jax.experimental.pallas import tpu as pltpu
from jax.experimental.pallas import tpu_sc as plsc

N = 50000
E = 800000
NC = 2
NS = 16
CB = 64
NB = 6
OUTER = 131
KCH = NB * OUTER
EPT = KCH * CB
NP = 51200
RPT = NP // NS
ZCH = RPT // CB

_mesh = plsc.VectorSubcoreMesh(core_axis_name="c", subcore_axis_name="s")


def _seg_body(upk4, srcp0, srcp1, dstp, zc, out, acc, sidx, didx, rows,
              isem, gsem, ssem):
    c = lax.axis_index("c")
    s = lax.axis_index("s")
    r0 = s * RPT

    pltpu.sync_copy(zc, rows.at[0, 0])

    def zbody(j, _):
        pltpu.sync_copy(rows.at[0, 0], acc.at[pl.ds(r0 + j * CB, CB)])
        return 0
    lax.fori_loop(0, ZCH, zbody, 0)

    @pl.when(c == 0)
    def _():
        pltpu.async_copy(srcp0.at[pl.ds(0, NB), s], sidx.at[0], isem)

    @pl.when(c == 1)
    def _():
        pltpu.async_copy(srcp1.at[pl.ds(0, NB), s], sidx.at[0], isem)
    pltpu.async_copy(dstp.at[pl.ds(0, NB), s], didx.at[0], isem)
    plsc.subcore_barrier()

    def edge_pipeline(srcp):
        def gbody(g, _):
            pb = lax.rem(g, 2)
            ic = lax.rem(g, 3)
            inx = lax.rem(g + 1, 3)

            @pl.when(g >= 2)
            def _():
                for b in range(NB):
                    pltpu.make_async_copy(zc, rows.at[pb, b], ssem).wait()

            @pl.when(g + 1 < OUTER)
            def _():
                pltpu.async_copy(srcp.at[pl.ds((g + 1) * NB, NB), s],
                                 sidx.at[inx], isem)
                pltpu.async_copy(dstp.at[pl.ds((g + 1) * NB, NB), s],
                                 didx.at[inx], isem)

            pltpu.make_async_copy(srcp.at[pl.ds(g * NB, NB), s],
                                  sidx.at[ic], isem).wait()
            pltpu.make_async_copy(dstp.at[pl.ds(g * NB, NB), s],
                                  didx.at[ic], isem).wait()

            for b in range(NB):
                pltpu.async_copy(upk4.at[sidx.at[ic, b]], rows.at[pb, b],
                                 gsem)
            for b in range(NB):
                pltpu.make_async_copy(upk4.at[sidx.at[ic, b]],
                                      rows.at[pb, b], gsem).wait()
                pltpu.async_copy(rows.at[pb, b], acc.at[didx.at[ic, b]],
                                 ssem, add=True)
            return 0
        lax.fori_loop(0, OUTER, gbody, 0)
        for pb in range(2):
            for b in range(NB):
                pltpu.make_async_copy(zc, rows.at[pb, b], ssem).wait()

    @pl.when(c == 0)
    def _():
        edge_pipeline(srcp0)

    @pl.when(c == 1)
    def _():
        edge_pipeline(srcp1)

    plsc.subcore_barrier()

    @pl.when(c == 0)
    def _():
        pltpu.sync_copy(acc.at[pl.ds(r0, RPT)],
                        out.at[pl.ds(r0, RPT), pl.ds(0, 32)])

    @pl.when(c == 1)
    def _():
        pltpu.sync_copy(acc.at[pl.ds(r0, RPT)],
                        out.at[pl.ds(r0, RPT), pl.ds(32, 32)])


_sc_segsum = pl.kernel(
    _seg_body,
    out_type=jax.ShapeDtypeStruct((NP, 128), jnp.float32),
    mesh=_mesh,
    compiler_params=pltpu.CompilerParams(use_tc_tiling_on_sc=False),
    scratch_types=[
        pltpu.VMEM_SHARED((NP, 32), jnp.float32),
        pltpu.VMEM((3, NB, CB), jnp.int32),
        pltpu.VMEM((3, NB, CB), jnp.int32),
        pltpu.VMEM((2, NB, CB, 32), jnp.float32),
        pltpu.SemaphoreType.DMA,
        pltpu.SemaphoreType.DMA,
        pltpu.SemaphoreType.DMA,
    ],
)


def _deg_body(dstp, oc, zc, out, acc, didx, onesv, zbuf, isem, ssem):
    c = lax.axis_index("c")
    s = lax.axis_index("s")

    @pl.when(c == 0)
    def _():
        pltpu.sync_copy(zc.at[:, pl.ds(0, 16)], zbuf)

        def zbody(j, _):
            pltpu.sync_copy(zbuf, acc.at[pl.ds(s * RPT + j * CB, CB)])
            return 0
        lax.fori_loop(0, ZCH, zbody, 0)
        pltpu.sync_copy(oc.at[:, pl.ds(0, 16)], onesv)
        pltpu.async_copy(dstp.at[pl.ds(0, NB), s], didx.at[0], isem)
        plsc.subcore_barrier()

        def gbody(g, _):
            ic = lax.rem(g, 3)
            inx = lax.rem(g + 1, 3)

            @pl.when(g >= 2)
            def _():
                for b in range(NB):
                    pltpu.make_async_copy(zc.at[:, pl.ds(0, 16)], onesv,
                                          ssem).wait()

            @pl.when(g + 1 < OUTER)
            def _():
                pltpu.async_copy(dstp.at[pl.ds((g + 1) * NB, NB), s],
                                 didx.at[inx], isem)

            pltpu.make_async_copy(dstp.at[pl.ds(g * NB, NB), s],
                                  didx.at[ic], isem).wait()

            for b in range(NB):
                pltpu.async_copy(onesv, acc.at[didx.at[ic, b]], ssem,
                                 add=True)
            return 0
        lax.fori_loop(0, OUTER, gbody, 0)
        for _pb in range(2):
            for b in range(NB):
                pltpu.make_async_copy(zc.at[:, pl.ds(0, 16)], onesv,
                                      ssem).wait()
        plsc.subcore_barrier()
        pltpu.sync_copy(acc.at[pl.ds(s * RPT, RPT)],
                        out.at[pl.ds(s * RPT, RPT), pl.ds(0, 16)])


_sc_deg = pl.kernel(
    _deg_body,
    out_type=jax.ShapeDtypeStruct((NP, 128), jnp.float32),
    mesh=_mesh,
    compiler_params=pltpu.CompilerParams(use_tc_tiling_on_sc=False),
    scratch_types=[
        pltpu.VMEM_SHARED((NP, 16), jnp.float32),
        pltpu.VMEM((3, NB, CB), jnp.int32),
        pltpu.VMEM((CB, 16), jnp.float32),
        pltpu.VMEM((CB, 16), jnp.float32),
        pltpu.SemaphoreType.DMA,
        pltpu.SemaphoreType.DMA,
    ],
)


R = 2000
G = N // R


def _row_spec(w):
    return pl.BlockSpec((R, w), lambda i: (i, 0))


def _const_spec(shape):
    return pl.BlockSpec(shape, lambda i: (0,) * len(shape))


def _xstats_body(x_ref, o_ref, acc):
    i = pl.program_id(0)

    @pl.when(i == 0)
    def _():
        acc[0] = 0.0
        acc[1] = 0.0

    xb = x_ref[...]
    acc[0] += jnp.sum(xb)
    acc[1] += jnp.sum(xb * xb)

    @pl.when(i == G - 1)
    def _():
        o_ref[0] = acc[0]
        o_ref[1] = acc[1]


def _xstats(x):
    return pl.pallas_call(
        _xstats_body,
        grid=(G,),
        in_specs=[_row_spec(1)],
        out_specs=pl.BlockSpec(memory_space=pltpu.SMEM),
        out_shape=jax.ShapeDtypeStruct((2,), jnp.float32),
        scratch_shapes=[pltpu.SMEM((2,), jnp.float32)],
    )(x)


def _dis_body(d_ref, o_ref):
    o_ref[...] = lax.rsqrt(d_ref[:, 0:1] + 1.0)


def _dis_from_counts(degp):
    return pl.pallas_call(
        _dis_body,
        grid=(8,),
        in_specs=[pl.BlockSpec((NP // 8, 128), lambda i: (i, 0))],
        out_specs=pl.BlockSpec((NP // 8, 1), lambda i: (i, 0)),
        out_shape=jax.ShapeDtypeStruct((NP, 1), jnp.float32),
    )(degp)


def _prelu(z, a):
    return jnp.where(z >= 0, z, a * z)


def _dense1_body(x_ref, a0_ref, b0_ref, al0_ref, w1_ref, b1_ref,
                 x1_ref, z1_ref, st_ref, acc):
    i = pl.program_id(0)

    @pl.when(i == 0)
    def _():
        acc[...] = jnp.zeros_like(acc)

    x1b = _prelu(x_ref[...] * a0_ref[...] + b0_ref[...], al0_ref[...])
    z1b = jnp.dot(x1b, w1_ref[...], preferred_element_type=jnp.float32)
    z1b = z1b + b1_ref[...]
    x1_ref[...] = x1b
    z1_ref[...] = z1b
    acc[0, :] += jnp.sum(z1b, axis=0)
    acc[1, :] += jnp.sum(z1b * z1b, axis=0)

    @pl.when(i == G - 1)
    def _():
        st_ref[...] = acc[...]


def _dense1(x, A0, B0, a0, W1, b1):
    return pl.pallas_call(
        _dense1_body,
        grid=(G,),
        in_specs=[_row_spec(1), _const_spec((1, 32)), _const_spec((1, 32)),
                  _const_spec((1, 32)), _const_spec((32, 64)),
                  _const_spec((1, 64))],
        out_specs=(_row_spec(32), _row_spec(64), _const_spec((2, 64))),
        out_shape=(jax.ShapeDtypeStruct((N, 32), jnp.float32),
                   jax.ShapeDtypeStruct((N, 64), jnp.float32),
                   jax.ShapeDtypeStruct((2, 64), jnp.float32)),
        scratch_shapes=[pltpu.VMEM((2, 64), jnp.float32)],
    )(x, A0, B0, a0, W1, b1)


def _init_body(z_ref, A_ref, B_ref, al_ref, dis_ref, x2_ref, upk_ref):
    x2b = _prelu(z_ref[...] * A_ref[...] + B_ref[...], al_ref[...])
    x2_ref[...] = x2b
    upk_ref[...] = jnp.concatenate([dis_ref[...] * x2b, 0.05 * x2b], axis=1)


def _ssg_init(z, A, B, a, dis):
    return pl.pallas_call(
        _init_body,
        grid=(G,),
        in_specs=[_row_spec(64), _const_spec((1, 64)), _const_spec((1, 64)),
                  _const_spec((1, 64)), _row_spec(1)],
        out_specs=(_row_spec(64), _row_spec(128)),
        out_shape=(jax.ShapeDtypeStruct((N, 64), jnp.float32),
                   jax.ShapeDtypeStruct((NP, 128), jnp.float32)),
    )(z, A, B, a, dis)


RS = NP // 8


def _step_body(ck, s_ref, upk_ref, dis_ref, o_ref):
    disb = dis_ref[...]
    u = upk_ref[:, :64]
    h = upk_ref[:, 64:]
    cur = disb * (s_ref[:, :64] + u)
    o_ref[...] = jnp.concatenate([disb * cur, h + ck * cur], axis=1)


def _prop_step(ck, spk, upk, dis):
    return pl.pallas_call(
        functools.partial(_step_body, ck),
        grid=(8,),
        in_specs=[pl.BlockSpec((RS, 128), lambda i: (i, 0)),
                  pl.BlockSpec((RS, 128), lambda i: (i, 0)),
                  pl.BlockSpec((RS, 1), lambda i: (i, 0))],
        out_specs=pl.BlockSpec((RS, 128), lambda i: (i, 0)),
        out_shape=jax.ShapeDtypeStruct((NP, 128), jnp.float32),
    )(spk, upk, dis)


def _mm_body(sl, a_ref, w_ref, b_ref, z_ref, st_ref, acc):
    i = pl.program_id(0)

    @pl.when(i == 0)
    def _():
        acc[...] = jnp.zeros_like(acc)

    ab = a_ref[:, sl[0]:sl[1]] if sl else a_ref[...]
    zb = jnp.dot(ab, w_ref[...], preferred_element_type=jnp.float32)
    zb = zb + b_ref[...]
    z_ref[...] = zb
    acc[0, :] += jnp.sum(zb, axis=0)
    acc[1, :] += jnp.sum(zb * zb, axis=0)

    @pl.when(i == G - 1)
    def _():
        st_ref[...] = acc[...]


def _mm_stats(a, W, b, from_h=False):
    fin, fout = W.shape
    aspec = _row_spec(128) if from_h else _row_spec(fin)
    sl = (64, 128) if from_h else None
    return pl.pallas_call(
        functools.partial(_mm_body, sl),
        grid=(G,),
        in_specs=[aspec, _const_spec((fin, fout)), _const_spec((1, fout))],
        out_specs=(_row_spec(fout), _const_spec((2, fout))),
        out_shape=(jax.ShapeDtypeStruct((N, fout), jnp.float32),
                   jax.ShapeDtypeStruct((2, fout), jnp.float32)),
        scratch_shapes=[pltpu.VMEM((2, fout), jnp.float32)],
    )(a, W, b)


def _post_body(z_ref, A_ref, B_ref, al_ref, xo_ref, p_ref, dis_ref,
               xn_ref, upk_ref):
    p = p_ref[0, 0]
    xn = _prelu(z_ref[...] * A_ref[...] + B_ref[...], al_ref[...])
    y = (1.0 - p) * xo_ref[...] + p * xn
    xn_ref[...] = xn
    upk_ref[...] = jnp.concatenate([dis_ref[...] * y, 0.05 * y], axis=1)


def _ssg_post(z, A, B, a, x_other, p, dis):
    return pl.pallas_call(
        _post_body,
        grid=(G,),
        in_specs=[_row_spec(64), _const_spec((1, 64)), _const_spec((1, 64)),
                  _const_spec((1, 64)), _row_spec(64), _const_spec((1, 1)),
                  _row_spec(1)],
        out_specs=(_row_spec(64), _row_spec(128)),
        out_shape=(jax.ShapeDtypeStruct((N, 64), jnp.float32),
                   jax.ShapeDtypeStruct((NP, 128), jnp.float32)),
    )(z, A, B, a, x_other, p, dis)


def _premix_body(z_ref, A_ref, B_ref, al_ref, x2_ref, x4_ref, w_ref, o_ref):
    x5 = _prelu(z_ref[...] * A_ref[...] + B_ref[...], al_ref[...])
    o_ref[...] = (w_ref[0, 0] * x2_ref[...] + w_ref[0, 1] * x4_ref[...]
                  + w_ref[0, 2] * x5)


def _premix(z, A, B, a, x2, x4, w):
    return pl.pallas_call(
        _premix_body,
        grid=(G,),
        in_specs=[_row_spec(64), _const_spec((1, 64)), _const_spec((1, 64)),
                  _const_spec((1, 64)), _row_spec(64), _row_spec(64),
                  _const_spec((1, 3))],
        out_specs=_row_spec(64),
        out_shape=jax.ShapeDtypeStruct((N, 64), jnp.float32),
    )(z, A, B, a, x2, x4, w)


def _final_body(z_ref, A_ref, B_ref, al_ref, x1_ref, p_ref, w6_ref, b6_ref,
                o_ref):
    p = p_ref[0, 0]
    x6 = _prelu(z_ref[...] * A_ref[...] + B_ref[...], al_ref[...])
    mix = (1.0 - p) * x1_ref[...] + p * x6
    o_ref[...] = jnp.dot(mix, w6_ref[...],
                         preferred_element_type=jnp.float32) + b6_ref[...]


def _final(z5, A, B, a, x1, p3, W6, b6):
    return pl.pallas_call(
        _final_body,
        grid=(G,),
        in_specs=[_row_spec(32), _const_spec((1, 32)), _const_spec((1, 32)),
                  _const_spec((1, 32)), _row_spec(32), _const_spec((1, 1)),
                  _const_spec((32, 8)), _const_spec((1, 8))],
        out_specs=_row_spec(8),
        out_shape=jax.ShapeDtypeStruct((N, 8), jnp.float32),
    )(z5, A, B, a, x1, p3, W6, b6)


def _bn_coeffs(st, g, be):
    mean = st[0] / N
    var = st[1] / N - mean * mean
    inv = g * lax.rsqrt(var + 1e-5)
    return inv, be - mean * inv


def _run_ssg(upk, dis, srcp0, srcp1, dst_t, zc, K):
    ck = 0.95 / K
    for _ in range(K):
        spk = _sc_segsum(upk.reshape(4 * NP, 32), srcp0, srcp1, dst_t, zc)
        upk = _prop_step(ck, spk, upk, dis)
    return upk


def kernel(x, adj_matrix, params):
    p = params
    pad = EPT * NS - E
    src = jnp.concatenate([adj_matrix[0], jnp.zeros((pad,), jnp.int32)])
    src4 = (src * 4).reshape(NS, KCH, CB).transpose(1, 0, 2)
    srcp0 = src4
    srcp1 = src4 + 1
    dst = jnp.concatenate([adj_matrix[1], jnp.full((pad,), N, jnp.int32)])
    dst = dst.reshape(NS, KCH, CB).transpose(1, 0, 2)
    zc128 = jnp.zeros((CB, 128), jnp.float32)
    oc128 = jnp.ones((CB, 128), jnp.float32)
    zc = jnp.zeros((CB, 32), jnp.float32)

    degp = _sc_deg(dst, oc128, zc128)
    dis = _dis_from_counts(degp)

    st = _xstats(x)
    mx, sx = st[0] / N, st[1] / N
    vx = sx - mx * mx
    w0 = p['W0'][0]
    mean0 = mx * w0 + p['b0']
    var0 = vx * w0 * w0
    inv0 = p['g0'] * lax.rsqrt(var0 + 1e-5)
    A0 = (w0 * inv0).reshape(1, 32)
    B0 = ((p['b0'] - mean0) * inv0 + p['be0']).reshape(1, 32)

    x1, z1, st1 = _dense1(x, A0, B0, p['a0'].reshape(1, 32),
                          p['W1'], p['b1'].reshape(1, 64))
    A1, B1 = _bn_coeffs(st1, p['g1'], p['be1'])

    x2, upk = _ssg_init(z1, A1.reshape(1, 64), B1.reshape(1, 64),
                        p['a1'].reshape(1, 64), dis)
    upk = _run_ssg(upk, dis, srcp0, srcp1, dst, zc, 3)
    z2, st2 = _mm_stats(upk, p['W2'], p['b2'].reshape(1, 64), from_h=True)
    A2, B2 = _bn_coeffs(st2, p['g2'], p['be2'])

    x3, upk = _ssg_post(z2, A2.reshape(1, 64), B2.reshape(1, 64),
                        p['a2'].reshape(1, 64), x2,
                        p['p0'].reshape(1, 1), dis)
    upk = _run_ssg(upk, dis, srcp0, srcp1, dst, zc, 4)
    z3, st3 = _mm_stats(upk, p['W3'], p['b3'].reshape(1, 64), from_h=True)
    A3, B3 = _bn_coeffs(st3, p['g3'], p['be3'])

    x4, upk = _ssg_post(z3, A3.reshape(1, 64), B3.reshape(1, 64),
                        p['a3'].reshape(1, 64), x3,
                        p['p1'].reshape(1, 1), dis)
    upk = _run_ssg(upk, dis, srcp0, srcp1, dst, zc, 3)
    z4, st4 = _mm_stats(upk, p['W4'], p['b4'].reshape(1, 64), from_h=True)
    A4, B4 = _bn_coeffs(st4, p['g4'], p['be4'])

    wmix = jax.nn.softmax(p['p2']).reshape(1, 3)
    pm = _premix(z4, A4.reshape(1, 64), B4.reshape(1, 64),
                 p['a4'].reshape(1, 64), x2, x4, wmix)
    z5, st5 = _mm_stats(pm, p['W5'], p['b5'].reshape(1, 32))
    A5, B5 = _bn_coeffs(st5, p['g5'], p['be5'])

    return _final(z5, A5.reshape(1, 32), B5.reshape(1, 32),
                  p['a5'].reshape(1, 32), x1, p['p3'].reshape(1, 1),
                  p['W6'], p['b6'].reshape(1, 8))


# speedup vs baseline: 1.0131x; 1.0131x over previous
import functools

import jax
import jax.numpy as jnp
from jax import lax
from jax.experimental import pallas as pl
from jax.experimental.pallas import tpu as pltpu
from jax.experimental.pallas import tpu_sc as plsc

N = 50000
E = 800000
NC = 2
NS = 16
CB = 128
NB = 3
OUTER = 131
KCH = NB * OUTER
EPT = KCH * CB
NP = 51200
RPT = NP // NS
ZCH = RPT // CB

_mesh = plsc.VectorSubcoreMesh(core_axis_name="c", subcore_axis_name="s")


def _seg_body(upk4, srcp0, srcp1, dstp, zc, out, acc, sidx, didx, rows,
              isem, gsem, ssem):
    c = lax.axis_index("c")
    s = lax.axis_index("s")
    r0 = s * RPT

    pltpu.sync_copy(zc, rows.at[0, 0])

    def zbody(j, _):
        pltpu.sync_copy(rows.at[0, 0], acc.at[pl.ds(r0 + j * CB, CB)])
        return 0
    lax.fori_loop(0, ZCH, zbody, 0)

    @pl.when(c == 0)
    def _():
        pltpu.async_copy(srcp0.at[pl.ds(0, NB), s], sidx.at[0], isem)

    @pl.when(c == 1)
    def _():
        pltpu.async_copy(srcp1.at[pl.ds(0, NB), s], sidx.at[0], isem)
    pltpu.async_copy(dstp.at[pl.ds(0, NB), s], didx.at[0], isem)
    plsc.subcore_barrier()

    def edge_pipeline(srcp):
        def gbody(g, _):
            pb = lax.rem(g, 2)
            ic = lax.rem(g, 3)
            inx = lax.rem(g + 1, 3)

            @pl.when(g >= 2)
            def _():
                for b in range(NB):
                    pltpu.make_async_copy(zc, rows.at[pb, b], ssem).wait()

            @pl.when(g + 1 < OUTER)
            def _():
                pltpu.async_copy(srcp.at[pl.ds((g + 1) * NB, NB), s],
                                 sidx.at[inx], isem)
                pltpu.async_copy(dstp.at[pl.ds((g + 1) * NB, NB), s],
                                 didx.at[inx], isem)

            pltpu.make_async_copy(srcp.at[pl.ds(g * NB, NB), s],
                                  sidx.at[ic], isem).wait()
            pltpu.make_async_copy(dstp.at[pl.ds(g * NB, NB), s],
                                  didx.at[ic], isem).wait()

            for b in range(NB):
                pltpu.async_copy(upk4.at[sidx.at[ic, b]], rows.at[pb, b],
                                 gsem)
            for b in range(NB):
                pltpu.make_async_copy(upk4.at[sidx.at[ic, b]],
                                      rows.at[pb, b], gsem).wait()
                pltpu.async_copy(rows.at[pb, b], acc.at[didx.at[ic, b]],
                                 ssem, add=True)
            return 0
        lax.fori_loop(0, OUTER, gbody, 0)
        for pb in range(2):
            for b in range(NB):
                pltpu.make_async_copy(zc, rows.at[pb, b], ssem).wait()

    @pl.when(c == 0)
    def _():
        edge_pipeline(srcp0)

    @pl.when(c == 1)
    def _():
        edge_pipeline(srcp1)

    plsc.subcore_barrier()

    @pl.when(c == 0)
    def _():
        pltpu.sync_copy(acc.at[pl.ds(r0, RPT)],
                        out.at[pl.ds(r0, RPT), pl.ds(0, 32)])

    @pl.when(c == 1)
    def _():
        pltpu.sync_copy(acc.at[pl.ds(r0, RPT)],
                        out.at[pl.ds(r0, RPT), pl.ds(32, 32)])


_sc_segsum = pl.kernel(
    _seg_body,
    out_type=jax.ShapeDtypeStruct((NP, 128), jnp.float32),
    mesh=_mesh,
    compiler_params=pltpu.CompilerParams(use_tc_tiling_on_sc=False),
    scratch_types=[
        pltpu.VMEM_SHARED((NP, 32), jnp.float32),
        pltpu.VMEM((3, NB, CB), jnp.int32),
        pltpu.VMEM((3, NB, CB), jnp.int32),
        pltpu.VMEM((2, NB, CB, 32), jnp.float32),
        pltpu.SemaphoreType.DMA,
        pltpu.SemaphoreType.DMA,
        pltpu.SemaphoreType.DMA,
    ],
)


def _deg_body(dstp, oc, zc, out, acc, didx, onesv, zbuf, isem, ssem):
    c = lax.axis_index("c")
    s = lax.axis_index("s")

    @pl.when(c == 0)
    def _():
        pltpu.sync_copy(zc.at[:, pl.ds(0, 16)], zbuf)

        def zbody(j, _):
            pltpu.sync_copy(zbuf, acc.at[pl.ds(s * RPT + j * CB, CB)])
            return 0
        lax.fori_loop(0, ZCH, zbody, 0)
        pltpu.sync_copy(oc.at[:, pl.ds(0, 16)], onesv)
        pltpu.async_copy(dstp.at[pl.ds(0, NB), s], didx.at[0], isem)
        plsc.subcore_barrier()

        def gbody(g, _):
            ic = lax.rem(g, 3)
            inx = lax.rem(g + 1, 3)

            @pl.when(g >= 2)
            def _():
                for b in range(NB):
                    pltpu.make_async_copy(zc.at[:, pl.ds(0, 16)], onesv,
                                          ssem).wait()

            @pl.when(g + 1 < OUTER)
            def _():
                pltpu.async_copy(dstp.at[pl.ds((g + 1) * NB, NB), s],
                                 didx.at[inx], isem)

            pltpu.make_async_copy(dstp.at[pl.ds(g * NB, NB), s],
                                  didx.at[ic], isem).wait()

            for b in range(NB):
                pltpu.async_copy(onesv, acc.at[didx.at[ic, b]], ssem,
                                 add=True)
            return 0
        lax.fori_loop(0, OUTER, gbody, 0)
        for _pb in range(2):
            for b in range(NB):
                pltpu.make_async_copy(zc.at[:, pl.ds(0, 16)], onesv,
                                      ssem).wait()
        plsc.subcore_barrier()
        pltpu.sync_copy(acc.at[pl.ds(s * RPT, RPT)],
                        out.at[pl.ds(s * RPT, RPT), pl.ds(0, 16)])


_sc_deg = pl.kernel(
    _deg_body,
    out_type=jax.ShapeDtypeStruct((NP, 128), jnp.float32),
    mesh=_mesh,
    compiler_params=pltpu.CompilerParams(use_tc_tiling_on_sc=False),
    scratch_types=[
        pltpu.VMEM_SHARED((NP, 16), jnp.float32),
        pltpu.VMEM((3, NB, CB), jnp.int32),
        pltpu.VMEM((CB, 16), jnp.float32),
        pltpu.VMEM((CB, 16), jnp.float32),
        pltpu.SemaphoreType.DMA,
        pltpu.SemaphoreType.DMA,
    ],
)


R = 2000
G = N // R


def _row_spec(w):
    return pl.BlockSpec((R, w), lambda i: (i, 0))


def _const_spec(shape):
    return pl.BlockSpec(shape, lambda i: (0,) * len(shape))


def _xstats_body(x_ref, o_ref, acc):
    i = pl.program_id(0)

    @pl.when(i == 0)
    def _():
        acc[0] = 0.0
        acc[1] = 0.0

    xb = x_ref[...]
    acc[0] += jnp.sum(xb)
    acc[1] += jnp.sum(xb * xb)

    @pl.when(i == G - 1)
    def _():
        o_ref[0] = acc[0]
        o_ref[1] = acc[1]


def _xstats(x):
    return pl.pallas_call(
        _xstats_body,
        grid=(G,),
        in_specs=[_row_spec(1)],
        out_specs=pl.BlockSpec(memory_space=pltpu.SMEM),
        out_shape=jax.ShapeDtypeStruct((2,), jnp.float32),
        scratch_shapes=[pltpu.SMEM((2,), jnp.float32)],
    )(x)


def _dis_body(d_ref, o_ref):
    o_ref[...] = lax.rsqrt(d_ref[:, 0:1] + 1.0)


def _dis_from_counts(degp):
    return pl.pallas_call(
        _dis_body,
        grid=(8,),
        in_specs=[pl.BlockSpec((NP // 8, 128), lambda i: (i, 0))],
        out_specs=pl.BlockSpec((NP // 8, 1), lambda i: (i, 0)),
        out_shape=jax.ShapeDtypeStruct((NP, 1), jnp.float32),
    )(degp)


def _prelu(z, a):
    return jnp.where(z >= 0, z, a * z)


def _dense1_body(x_ref, a0_ref, b0_ref, al0_ref, w1_ref, b1_ref,
                 x1_ref, z1_ref, st_ref, acc):
    i = pl.program_id(0)

    @pl.when(i == 0)
    def _():
        acc[...] = jnp.zeros_like(acc)

    x1b = _prelu(x_ref[...] * a0_ref[...] + b0_ref[...], al0_ref[...])
    z1b = jnp.dot(x1b, w1_ref[...], preferred_element_type=jnp.float32)
    z1b = z1b + b1_ref[...]
    x1_ref[...] = x1b
    z1_ref[...] = z1b
    acc[0, :] += jnp.sum(z1b, axis=0)
    acc[1, :] += jnp.sum(z1b * z1b, axis=0)

    @pl.when(i == G - 1)
    def _():
        st_ref[...] = acc[...]


def _dense1(x, A0, B0, a0, W1, b1):
    return pl.pallas_call(
        _dense1_body,
        grid=(G,),
        in_specs=[_row_spec(1), _const_spec((1, 32)), _const_spec((1, 32)),
                  _const_spec((1, 32)), _const_spec((32, 64)),
                  _const_spec((1, 64))],
        out_specs=(_row_spec(32), _row_spec(64), _const_spec((2, 64))),
        out_shape=(jax.ShapeDtypeStruct((N, 32), jnp.float32),
                   jax.ShapeDtypeStruct((N, 64), jnp.float32),
                   jax.ShapeDtypeStruct((2, 64), jnp.float32)),
        scratch_shapes=[pltpu.VMEM((2, 64), jnp.float32)],
    )(x, A0, B0, a0, W1, b1)


def _init_body(z_ref, A_ref, B_ref, al_ref, dis_ref, x2_ref, upk_ref):
    x2b = _prelu(z_ref[...] * A_ref[...] + B_ref[...], al_ref[...])
    x2_ref[...] = x2b
    upk_ref[...] = jnp.concatenate([dis_ref[...] * x2b, 0.05 * x2b], axis=1)


def _ssg_init(z, A, B, a, dis):
    return pl.pallas_call(
        _init_body,
        grid=(G,),
        in_specs=[_row_spec(64), _const_spec((1, 64)), _const_spec((1, 64)),
                  _const_spec((1, 64)), _row_spec(1)],
        out_specs=(_row_spec(64), _row_spec(128)),
        out_shape=(jax.ShapeDtypeStruct((N, 64), jnp.float32),
                   jax.ShapeDtypeStruct((NP, 128), jnp.float32)),
    )(z, A, B, a, dis)


RS = NP // 8


def _step_body(ck, s_ref, upk_ref, dis_ref, o_ref):
    disb = dis_ref[...]
    u = upk_ref[:, :64]
    h = upk_ref[:, 64:]
    cur = disb * (s_ref[:, :64] + u)
    o_ref[...] = jnp.concatenate([disb * cur, h + ck * cur], axis=1)


def _prop_step(ck, spk, upk, dis):
    return pl.pallas_call(
        functools.partial(_step_body, ck),
        grid=(8,),
        in_specs=[pl.BlockSpec((RS, 128), lambda i: (i, 0)),
                  pl.BlockSpec((RS, 128), lambda i: (i, 0)),
                  pl.BlockSpec((RS, 1), lambda i: (i, 0))],
        out_specs=pl.BlockSpec((RS, 128), lambda i: (i, 0)),
        out_shape=jax.ShapeDtypeStruct((NP, 128), jnp.float32),
    )(spk, upk, dis)


def _mm_body(sl, a_ref, w_ref, b_ref, z_ref, st_ref, acc):
    i = pl.program_id(0)

    @pl.when(i == 0)
    def _():
        acc[...] = jnp.zeros_like(acc)

    ab = a_ref[:, sl[0]:sl[1]] if sl else a_ref[...]
    zb = jnp.dot(ab, w_ref[...], preferred_element_type=jnp.float32)
    zb = zb + b_ref[...]
    z_ref[...] = zb
    acc[0, :] += jnp.sum(zb, axis=0)
    acc[1, :] += jnp.sum(zb * zb, axis=0)

    @pl.when(i == G - 1)
    def _():
        st_ref[...] = acc[...]


def _mm_stats(a, W, b, from_h=False):
    fin, fout = W.shape
    aspec = _row_spec(128) if from_h else _row_spec(fin)
    sl = (64, 128) if from_h else None
    return pl.pallas_call(
        functools.partial(_mm_body, sl),
        grid=(G,),
        in_specs=[aspec, _const_spec((fin, fout)), _const_spec((1, fout))],
        out_specs=(_row_spec(fout), _const_spec((2, fout))),
        out_shape=(jax.ShapeDtypeStruct((N, fout), jnp.float32),
                   jax.ShapeDtypeStruct((2, fout), jnp.float32)),
        scratch_shapes=[pltpu.VMEM((2, fout), jnp.float32)],
    )(a, W, b)


def _post_body(z_ref, A_ref, B_ref, al_ref, xo_ref, p_ref, dis_ref,
               xn_ref, upk_ref):
    p = p_ref[0, 0]
    xn = _prelu(z_ref[...] * A_ref[...] + B_ref[...], al_ref[...])
    y = (1.0 - p) * xo_ref[...] + p * xn
    xn_ref[...] = xn
    upk_ref[...] = jnp.concatenate([dis_ref[...] * y, 0.05 * y], axis=1)


def _ssg_post(z, A, B, a, x_other, p, dis):
    return pl.pallas_call(
        _post_body,
        grid=(G,),
        in_specs=[_row_spec(64), _const_spec((1, 64)), _const_spec((1, 64)),
                  _const_spec((1, 64)), _row_spec(64), _const_spec((1, 1)),
                  _row_spec(1)],
        out_specs=(_row_spec(64), _row_spec(128)),
        out_shape=(jax.ShapeDtypeStruct((N, 64), jnp.float32),
                   jax.ShapeDtypeStruct((NP, 128), jnp.float32)),
    )(z, A, B, a, x_other, p, dis)


def _premix_body(z_ref, A_ref, B_ref, al_ref, x2_ref, x4_ref, w_ref, o_ref):
    x5 = _prelu(z_ref[...] * A_ref[...] + B_ref[...], al_ref[...])
    o_ref[...] = (w_ref[0, 0] * x2_ref[...] + w_ref[0, 1] * x4_ref[...]
                  + w_ref[0, 2] * x5)


def _premix(z, A, B, a, x2, x4, w):
    return pl.pallas_call(
        _premix_body,
        grid=(G,),
        in_specs=[_row_spec(64), _const_spec((1, 64)), _const_spec((1, 64)),
                  _const_spec((1, 64)), _row_spec(64), _row_spec(64),
                  _const_spec((1, 3))],
        out_specs=_row_spec(64),
        out_shape=jax.ShapeDtypeStruct((N, 64), jnp.float32),
    )(z, A, B, a, x2, x4, w)


def _final_body(z_ref, A_ref, B_ref, al_ref, x1_ref, p_ref, w6_ref, b6_ref,
                o_ref):
    p = p_ref[0, 0]
    x6 = _prelu(z_ref[...] * A_ref[...] + B_ref[...], al_ref[...])
    mix = (1.0 - p) * x1_ref[...] + p * x6
    o_ref[...] = jnp.dot(mix, w6_ref[...],
                         preferred_element_type=jnp.float32) + b6_ref[...]


def _final(z5, A, B, a, x1, p3, W6, b6):
    return pl.pallas_call(
        _final_body,
        grid=(G,),
        in_specs=[_row_spec(32), _const_spec((1, 32)), _const_spec((1, 32)),
                  _const_spec((1, 32)), _row_spec(32), _const_spec((1, 1)),
                  _const_spec((32, 8)), _const_spec((1, 8))],
        out_specs=_row_spec(8),
        out_shape=jax.ShapeDtypeStruct((N, 8), jnp.float32),
    )(z5, A, B, a, x1, p3, W6, b6)


def _bn_coeffs(st, g, be):
    mean = st[0] / N
    var = st[1] / N - mean * mean
    inv = g * lax.rsqrt(var + 1e-5)
    return inv, be - mean * inv


def _run_ssg(upk, dis, srcp0, srcp1, dst_t, zc, K):
    ck = 0.95 / K
    for _ in range(K):
        spk = _sc_segsum(upk.reshape(4 * NP, 32), srcp0, srcp1, dst_t, zc)
        upk = _prop_step(ck, spk, upk, dis)
    return upk


def kernel(x, adj_matrix, params):
    p = params
    pad = EPT * NS - E
    src = jnp.concatenate([adj_matrix[0], jnp.zeros((pad,), jnp.int32)])
    src4 = (src * 4).reshape(NS, KCH, CB).transpose(1, 0, 2)
    srcp0 = src4
    srcp1 = src4 + 1
    dst = jnp.concatenate([adj_matrix[1], jnp.full((pad,), N, jnp.int32)])
    dst = dst.reshape(NS, KCH, CB).transpose(1, 0, 2)
    zc128 = jnp.zeros((CB, 128), jnp.float32)
    oc128 = jnp.ones((CB, 128), jnp.float32)
    zc = jnp.zeros((CB, 32), jnp.float32)

    degp = _sc_deg(dst, oc128, zc128)
    dis = _dis_from_counts(degp)

    st = _xstats(x)
    mx, sx = st[0] / N, st[1] / N
    vx = sx - mx * mx
    w0 = p['W0'][0]
    mean0 = mx * w0 + p['b0']
    var0 = vx * w0 * w0
    inv0 = p['g0'] * lax.rsqrt(var0 + 1e-5)
    A0 = (w0 * inv0).reshape(1, 32)
    B0 = ((p['b0'] - mean0) * inv0 + p['be0']).reshape(1, 32)

    x1, z1, st1 = _dense1(x, A0, B0, p['a0'].reshape(1, 32),
                          p['W1'], p['b1'].reshape(1, 64))
    A1, B1 = _bn_coeffs(st1, p['g1'], p['be1'])

    x2, upk = _ssg_init(z1, A1.reshape(1, 64), B1.reshape(1, 64),
                        p['a1'].reshape(1, 64), dis)
    upk = _run_ssg(upk, dis, srcp0, srcp1, dst, zc, 3)
    z2, st2 = _mm_stats(upk, p['W2'], p['b2'].reshape(1, 64), from_h=True)
    A2, B2 = _bn_coeffs(st2, p['g2'], p['be2'])

    x3, upk = _ssg_post(z2, A2.reshape(1, 64), B2.reshape(1, 64),
                        p['a2'].reshape(1, 64), x2,
                        p['p0'].reshape(1, 1), dis)
    upk = _run_ssg(upk, dis, srcp0, srcp1, dst, zc, 4)
    z3, st3 = _mm_stats(upk, p['W3'], p['b3'].reshape(1, 64), from_h=True)
    A3, B3 = _bn_coeffs(st3, p['g3'], p['be3'])

    x4, upk = _ssg_post(z3, A3.reshape(1, 64), B3.reshape(1, 64),
                        p['a3'].reshape(1, 64), x3,
                        p['p1'].reshape(1, 1), dis)
    upk = _run_ssg(upk, dis, srcp0, srcp1, dst, zc, 3)
    z4, st4 = _mm_stats(upk, p['W4'], p['b4'].reshape(1, 64), from_h=True)
    A4, B4 = _bn_coeffs(st4, p['g4'], p['be4'])

    wmix = jax.nn.softmax(p['p2']).reshape(1, 3)
    pm = _premix(z4, A4.reshape(1, 64), B4.reshape(1, 64),
                 p['a4'].reshape(1, 64), x2, x4, wmix)
    z5, st5 = _mm_stats(pm, p['W5'], p['b5'].reshape(1, 32))
    A5, B5 = _bn_coeffs(st5, p['g5'], p['be5'])

    return _final(z5, A5.reshape(1, 32), B5.reshape(1, 32),
                  p['a5'].reshape(1, 32), x1, p['p3'].reshape(1, 1),
                  p['W6'], p['b6'].reshape(1, 8))
